# baseline (speedup 1.0000x reference)
import jax
import jax.numpy as jnp
from jax import lax
from jax.experimental import pallas as pl
from jax.experimental.pallas import tpu as pltpu
from jax.experimental.pallas import tpu_sc as plsc

_B = 16
_BSC = 2
_N = 2048
_NW = 32
_QS = _N // _NW
_NC = _N // 16


def _lane_bcast(v, u):
    idx = jnp.full((16,), u, jnp.int32)
    return v.at[idx].get(mode="promise_in_bounds")


def _hmin(v, lane):
    for sh in (8, 4, 2, 1):
        idx = jnp.bitwise_xor(lane, jnp.int32(sh))
        v = jnp.minimum(v, v.at[idx].get(mode="promise_in_bounds"))
    return v


def _round_bf16(x):
    i = lax.bitcast_convert_type(x, jnp.int32)
    lsb = jnp.bitwise_and(lax.shift_right_logical(i, 16), jnp.int32(1))
    r = i + jnp.int32(0x7FFF) + lsb
    r = jnp.bitwise_and(r, jnp.int32(-65536))
    return lax.bitcast_convert_type(r, jnp.float32)


def _vsqrt(x):
    xh = x * 0.5
    i = lax.bitcast_convert_type(x, jnp.int32)
    i = jnp.int32(0x5F3759DF) - lax.shift_right_logical(i, 1)
    y = lax.bitcast_convert_type(i, jnp.float32)
    for _ in range(3):
        y = y * (1.5 - xh * y * y)
    return x * y


def _gat(v, idx):
    return v.at[idx].get(mode="promise_in_bounds")


def _coords(raw_v, p0, lane):
    w0 = p0 * 3
    v0 = raw_v[pl.ds(w0, 16)]
    v1 = raw_v[pl.ds(w0 + 16, 16)]
    v2 = raw_v[pl.ds(w0 + 32, 16)]
    out = []
    for c in range(3):
        flat = lane * 3 + c
        lidx = jnp.bitwise_and(flat, jnp.int32(15))
        src = lax.shift_right_logical(flat, 4)
        g = jnp.where(src == 0, _gat(v0, lidx),
                      jnp.where(src == 1, _gat(v1, lidx), _gat(v2, lidx)))
        out.append(g)
    return out[0], out[1], out[2]


def _prep(raw_v, pre_v, lane):
    def body(c, _):
        sl = pl.ds(c * 16, 16)
        x, y, z = _coords(raw_v, c * 16, lane)
        pre_v[0, sl] = _round_bf16(x) * -2.0
        pre_v[1, sl] = _round_bf16(y) * -2.0
        pre_v[2, sl] = _round_bf16(z) * -2.0
        pre_v[3, sl] = (x * x + y * y) + z * z
        return 0
    lax.fori_loop(0, _NC, body, 0)


def _pass(pre_v, qry_v, base, acc):
    lane = lax.iota(jnp.int32, 16)

    def qblock(qb, acc):
        qxv, qyv, qzv = _coords(qry_v, base + qb * 16, lane)
        nqv = (qxv * qxv + qyv * qyv) + qzv * qzv
        qxb = _round_bf16(qxv)
        qyb = _round_bf16(qyv)
        qzb = _round_bf16(qzv)

        packed = jnp.zeros((16,), jnp.float32)
        for u0 in range(0, 16, 2):
            qs = []
            for u in (u0, u0 + 1):
                qs.append((_lane_bcast(qxb, u), _lane_bcast(qyb, u),
                           _lane_bcast(qzb, u), _lane_bcast(nqv, u)))

            def cbody(c, ms):
                csl = pl.ds(c * 16, 16)
                rx = pre_v[0, csl]
                ry = pre_v[1, csl]
                rz = pre_v[2, csl]
                nr = pre_v[3, csl]
                out = []
                for (qx, qy, qz, nq), m in zip(qs, ms):
                    s = (rx * qx + ry * qy) + rz * qz
                    d2 = (nq + nr) + s
                    out.append(jnp.minimum(m, d2))
                return tuple(out)

            init = (jnp.full((16,), 1e30, jnp.float32),
                    jnp.full((16,), 1e30, jnp.float32))
            ms = lax.fori_loop(0, _NC, cbody, init)
            for i, u in enumerate((u0, u0 + 1)):
                packed = jnp.where(lane == u, _hmin(ms[i], lane), packed)
        return acc + _vsqrt(jnp.maximum(packed, 1e-12))

    return lax.fori_loop(0, _QS // 16, qblock, acc)


def _sc_chamfer(a_hbm, b_hbm, out_hbm, a_v, b_v, pa_v, pb_v, acc_v):
    wid = lax.axis_index("s") * 2 + lax.axis_index("c")
    base = wid * _QS
    lane = lax.iota(jnp.int32, 16)

    def batch_body(k, acc):
        pltpu.sync_copy(a_hbm.at[k], a_v)
        pltpu.sync_copy(b_hbm.at[k], b_v)
        _prep(a_v, pa_v, lane)
        _prep(b_v, pb_v, lane)
        acc = _pass(pa_v, b_v, base, acc)
        acc = _pass(pb_v, a_v, base, acc)
        return acc

    acc = lax.fori_loop(0, _BSC, batch_body, jnp.zeros((16,), jnp.float32))
    acc_v[...] = acc * jnp.float32(1.0 / _N)
    pltpu.sync_copy(acc_v, out_hbm.at[wid])


def _sc_part(at, bt):
    mesh = plsc.VectorSubcoreMesh(core_axis_name="c", subcore_axis_name="s")
    return pl.kernel(
        _sc_chamfer,
        out_type=jax.ShapeDtypeStruct((_NW, 16), jnp.float32),
        mesh=mesh,
        scratch_types=[
            pltpu.VMEM((_N * 3,), jnp.float32),
            pltpu.VMEM((_N * 3,), jnp.float32),
            pltpu.VMEM((4, _N), jnp.float32),
            pltpu.VMEM((4, _N), jnp.float32),
            pltpu.VMEM((16,), jnp.float32),
        ],
    )(at, bt)


def _tc_body(a_ref, b_ref, out_ref):
    a = a_ref[0]
    b = b_ref[0]
    ab2 = lax.dot_general(a * -2.0, b, (((1,), (1,)), ((), ())),
                          preferred_element_type=jnp.float32)
    na = jnp.sum(a * a, axis=1)
    nb = jnp.sum(b * b, axis=1)
    d2 = (na[:, None] + ab2) + nb[None, :]
    m_b = jnp.min(d2, axis=0)
    m_a = jnp.min(d2, axis=1)
    loss = (jnp.mean(jnp.sqrt(jnp.maximum(m_b, 1e-12)))
            + jnp.mean(jnp.sqrt(jnp.maximum(m_a, 1e-12))))
    out_ref[...] = jnp.full((1, 1, 128), loss, jnp.float32)


def _tc_part(a, b):
    nb = a.shape[0]
    losses = pl.pallas_call(
        _tc_body,
        grid=(nb,),
        in_specs=[
            pl.BlockSpec((1, _N, 3), lambda i: (i, 0, 0)),
            pl.BlockSpec((1, _N, 3), lambda i: (i, 0, 0)),
        ],
        out_specs=pl.BlockSpec((1, 1, 128), lambda i: (i, 0, 0)),
        out_shape=jax.ShapeDtypeStruct((nb, 1, 128), jnp.float32),
    )(a, b)
    return jnp.sum(losses[:, 0, 0])


@jax.jit
def kernel(input, target):
    sc_out = _sc_part(jnp.reshape(input, (_B, _N * 3)),
                      jnp.reshape(target, (_B, _N * 3)))
    tc_loss = _tc_part(input[_BSC:], target[_BSC:])
    return jnp.reshape(jnp.sum(sc_out) + tc_loss, (1,))


# speedup vs baseline: 1.1900x; 1.1900x over previous
import jax
import jax.numpy as jnp
from jax import lax
from jax.experimental import pallas as pl
from jax.experimental.pallas import tpu as pltpu
from jax.experimental.pallas import tpu_sc as plsc

_B = 16
_BSC = 1
_N = 2048
_NW = 32
_QS = _N // _NW
_NC = _N // 16


def _lane_bcast(v, u):
    idx = jnp.full((16,), u, jnp.int32)
    return v.at[idx].get(mode="promise_in_bounds")


def _hmin(v, lane):
    for sh in (8, 4, 2, 1):
        idx = jnp.bitwise_xor(lane, jnp.int32(sh))
        v = jnp.minimum(v, v.at[idx].get(mode="promise_in_bounds"))
    return v


def _round_bf16(x):
    i = lax.bitcast_convert_type(x, jnp.int32)
    lsb = jnp.bitwise_and(lax.shift_right_logical(i, 16), jnp.int32(1))
    r = i + jnp.int32(0x7FFF) + lsb
    r = jnp.bitwise_and(r, jnp.int32(-65536))
    return lax.bitcast_convert_type(r, jnp.float32)


def _vsqrt(x):
    xh = x * 0.5
    i = lax.bitcast_convert_type(x, jnp.int32)
    i = jnp.int32(0x5F3759DF) - lax.shift_right_logical(i, 1)
    y = lax.bitcast_convert_type(i, jnp.float32)
    for _ in range(3):
        y = y * (1.5 - xh * y * y)
    return x * y


def _gat(v, idx):
    return v.at[idx].get(mode="promise_in_bounds")


def _coords(raw_v, p0, lane):
    w0 = p0 * 3
    v0 = raw_v[pl.ds(w0, 16)]
    v1 = raw_v[pl.ds(w0 + 16, 16)]
    v2 = raw_v[pl.ds(w0 + 32, 16)]
    out = []
    for c in range(3):
        flat = lane * 3 + c
        lidx = jnp.bitwise_and(flat, jnp.int32(15))
        src = lax.shift_right_logical(flat, 4)
        g = jnp.where(src == 0, _gat(v0, lidx),
                      jnp.where(src == 1, _gat(v1, lidx), _gat(v2, lidx)))
        out.append(g)
    return out[0], out[1], out[2]


def _prep(raw_v, pre_v, lane):
    def body(c, _):
        sl = pl.ds(c * 16, 16)
        x, y, z = _coords(raw_v, c * 16, lane)
        pre_v[0, sl] = _round_bf16(x) * -2.0
        pre_v[1, sl] = _round_bf16(y) * -2.0
        pre_v[2, sl] = _round_bf16(z) * -2.0
        pre_v[3, sl] = (x * x + y * y) + z * z
        return 0
    lax.fori_loop(0, _NC, body, 0)


def _pass(pre_v, qry_v, base, acc):
    lane = lax.iota(jnp.int32, 16)

    def qblock(qb, acc):
        qxv, qyv, qzv = _coords(qry_v, base + qb * 16, lane)
        nqv = (qxv * qxv + qyv * qyv) + qzv * qzv
        qxb = _round_bf16(qxv)
        qyb = _round_bf16(qyv)
        qzb = _round_bf16(qzv)

        packed = jnp.zeros((16,), jnp.float32)
        for u0 in range(0, 16, 2):
            qs = []
            for u in (u0, u0 + 1):
                qs.append((_lane_bcast(qxb, u), _lane_bcast(qyb, u),
                           _lane_bcast(qzb, u), _lane_bcast(nqv, u)))

            def cbody(c, ms):
                csl = pl.ds(c * 16, 16)
                rx = pre_v[0, csl]
                ry = pre_v[1, csl]
                rz = pre_v[2, csl]
                nr = pre_v[3, csl]
                out = []
                for (qx, qy, qz, nq), m in zip(qs, ms):
                    s = (rx * qx + ry * qy) + rz * qz
                    d2 = (nq + nr) + s
                    out.append(jnp.minimum(m, d2))
                return tuple(out)

            init = (jnp.full((16,), 1e30, jnp.float32),
                    jnp.full((16,), 1e30, jnp.float32))
            ms = lax.fori_loop(0, _NC, cbody, init)
            for i, u in enumerate((u0, u0 + 1)):
                packed = jnp.where(lane == u, _hmin(ms[i], lane), packed)
        return acc + _vsqrt(jnp.maximum(packed, 1e-12))

    return lax.fori_loop(0, _QS // 16, qblock, acc)


def _sc_chamfer(a_hbm, b_hbm, out_hbm, a_v, b_v, pa_v, pb_v, acc_v):
    wid = lax.axis_index("s") * 2 + lax.axis_index("c")
    base = wid * _QS
    lane = lax.iota(jnp.int32, 16)

    def batch_body(k, acc):
        pltpu.sync_copy(a_hbm.at[k], a_v)
        pltpu.sync_copy(b_hbm.at[k], b_v)
        _prep(a_v, pa_v, lane)
        _prep(b_v, pb_v, lane)
        acc = _pass(pa_v, b_v, base, acc)
        acc = _pass(pb_v, a_v, base, acc)
        return acc

    acc = lax.fori_loop(0, _BSC, batch_body, jnp.zeros((16,), jnp.float32))
    acc_v[...] = acc * jnp.float32(1.0 / _N)
    pltpu.sync_copy(acc_v, out_hbm.at[wid])


def _sc_part(at, bt):
    mesh = plsc.VectorSubcoreMesh(core_axis_name="c", subcore_axis_name="s")
    return pl.kernel(
        _sc_chamfer,
        out_type=jax.ShapeDtypeStruct((_NW, 16), jnp.float32),
        mesh=mesh,
        scratch_types=[
            pltpu.VMEM((_N * 3,), jnp.float32),
            pltpu.VMEM((_N * 3,), jnp.float32),
            pltpu.VMEM((4, _N), jnp.float32),
            pltpu.VMEM((4, _N), jnp.float32),
            pltpu.VMEM((16,), jnp.float32),
        ],
    )(at, bt)


def _tc_body(a_ref, b_ref, out_ref):
    a = a_ref[0]
    b = b_ref[0]
    ab2 = lax.dot_general(a * -2.0, b, (((1,), (1,)), ((), ())),
                          preferred_element_type=jnp.float32)
    na = jnp.sum(a * a, axis=1)
    nb = jnp.sum(b * b, axis=1)
    d2 = (na[:, None] + ab2) + nb[None, :]
    m_b = jnp.min(d2, axis=0)
    m_a = jnp.min(d2, axis=1)
    loss = (jnp.mean(jnp.sqrt(jnp.maximum(m_b, 1e-12)))
            + jnp.mean(jnp.sqrt(jnp.maximum(m_a, 1e-12))))
    out_ref[...] = jnp.full((1, 1, 128), loss, jnp.float32)


def _tc_part(a, b):
    nb = a.shape[0]
    losses = pl.pallas_call(
        _tc_body,
        grid=(nb,),
        in_specs=[
            pl.BlockSpec((1, _N, 3), lambda i: (i, 0, 0)),
            pl.BlockSpec((1, _N, 3), lambda i: (i, 0, 0)),
        ],
        out_specs=pl.BlockSpec((1, 1, 128), lambda i: (i, 0, 0)),
        out_shape=jax.ShapeDtypeStruct((nb, 1, 128), jnp.float32),
    )(a, b)
    return jnp.sum(losses[:, 0, 0])


@jax.jit
def kernel(input, target):
    sc_out = _sc_part(jnp.reshape(input[:_BSC], (_BSC, _N * 3)),
                      jnp.reshape(target[:_BSC], (_BSC, _N * 3)))
    tc_loss = _tc_part(input[_BSC:], target[_BSC:])
    return jnp.reshape(jnp.sum(sc_out) + tc_loss, (1,))


# speedup vs baseline: 1.2226x; 1.0274x over previous
import jax
import jax.numpy as jnp
from jax import lax
from jax.experimental import pallas as pl
from jax.experimental.pallas import tpu as pltpu
from jax.experimental.pallas import tpu_sc as plsc

_B = 16
_BSC = 1
_N = 2048
_NW = 32
_QS = _N // _NW
_NC = _N // 16


def _lane_bcast(v, u):
    idx = jnp.full((16,), u, jnp.int32)
    return v.at[idx].get(mode="promise_in_bounds")


def _hmin(v, lane):
    for sh in (8, 4, 2, 1):
        idx = jnp.bitwise_xor(lane, jnp.int32(sh))
        v = jnp.minimum(v, v.at[idx].get(mode="promise_in_bounds"))
    return v


def _round_bf16(x):
    i = lax.bitcast_convert_type(x, jnp.int32)
    lsb = jnp.bitwise_and(lax.shift_right_logical(i, 16), jnp.int32(1))
    r = i + jnp.int32(0x7FFF) + lsb
    r = jnp.bitwise_and(r, jnp.int32(-65536))
    return lax.bitcast_convert_type(r, jnp.float32)


def _vsqrt(x):
    xh = x * 0.5
    i = lax.bitcast_convert_type(x, jnp.int32)
    i = jnp.int32(0x5F3759DF) - lax.shift_right_logical(i, 1)
    y = lax.bitcast_convert_type(i, jnp.float32)
    for _ in range(3):
        y = y * (1.5 - xh * y * y)
    return x * y


def _gat(v, idx):
    return v.at[idx].get(mode="promise_in_bounds")


def _coords(raw_v, p0, lane):
    w0 = p0 * 3
    v0 = raw_v[pl.ds(w0, 16)]
    v1 = raw_v[pl.ds(w0 + 16, 16)]
    v2 = raw_v[pl.ds(w0 + 32, 16)]
    out = []
    for c in range(3):
        flat = lane * 3 + c
        lidx = jnp.bitwise_and(flat, jnp.int32(15))
        src = lax.shift_right_logical(flat, 4)
        g = jnp.where(src == 0, _gat(v0, lidx),
                      jnp.where(src == 1, _gat(v1, lidx), _gat(v2, lidx)))
        out.append(g)
    return out[0], out[1], out[2]


def _prep(raw_v, pre_v, lane):
    def body(c, _):
        sl = pl.ds(c * 16, 16)
        x, y, z = _coords(raw_v, c * 16, lane)
        pre_v[0, sl] = _round_bf16(x) * -2.0
        pre_v[1, sl] = _round_bf16(y) * -2.0
        pre_v[2, sl] = _round_bf16(z) * -2.0
        pre_v[3, sl] = (x * x + y * y) + z * z
        return 0
    lax.fori_loop(0, _NC, body, 0)


def _pass(pre_v, qry_v, base, acc):
    lane = lax.iota(jnp.int32, 16)

    def qblock(qb, acc):
        qxv, qyv, qzv = _coords(qry_v, base + qb * 16, lane)
        nqv = (qxv * qxv + qyv * qyv) + qzv * qzv
        qxb = _round_bf16(qxv)
        qyb = _round_bf16(qyv)
        qzb = _round_bf16(qzv)

        packed = jnp.zeros((16,), jnp.float32)
        for u0 in range(0, 16, 2):
            qs = []
            for u in (u0, u0 + 1):
                qs.append((_lane_bcast(qxb, u), _lane_bcast(qyb, u),
                           _lane_bcast(qzb, u), _lane_bcast(nqv, u)))

            def cbody(c, ms):
                csl = pl.ds(c * 16, 16)
                rx = pre_v[0, csl]
                ry = pre_v[1, csl]
                rz = pre_v[2, csl]
                nr = pre_v[3, csl]
                out = []
                for (qx, qy, qz, nq), m in zip(qs, ms):
                    s = (rx * qx + ry * qy) + rz * qz
                    d2 = (nq + nr) + s
                    out.append(jnp.minimum(m, d2))
                return tuple(out)

            init = (jnp.full((16,), 1e30, jnp.float32),
                    jnp.full((16,), 1e30, jnp.float32))
            ms = lax.fori_loop(0, _NC, cbody, init)
            for i, u in enumerate((u0, u0 + 1)):
                packed = jnp.where(lane == u, _hmin(ms[i], lane), packed)
        return acc + _vsqrt(jnp.maximum(packed, 1e-12))

    return lax.fori_loop(0, _QS // 16, qblock, acc)


def _sc_chamfer(ab_hbm, out_hbm, a_v, b_v, pa_v, pb_v, acc_v):
    wid = lax.axis_index("s") * 2 + lax.axis_index("c")
    base = wid * _QS
    lane = lax.iota(jnp.int32, 16)

    def batch_body(k, acc):
        pltpu.sync_copy(ab_hbm.at[k], a_v)
        pltpu.sync_copy(ab_hbm.at[_BSC + k], b_v)
        _prep(a_v, pa_v, lane)
        _prep(b_v, pb_v, lane)
        acc = _pass(pa_v, b_v, base, acc)
        acc = _pass(pb_v, a_v, base, acc)
        return acc

    acc = lax.fori_loop(0, _BSC, batch_body, jnp.zeros((16,), jnp.float32))
    acc_v[...] = acc * jnp.float32(1.0 / _N)
    pltpu.sync_copy(acc_v, out_hbm.at[wid])


def _sc_part(ab):
    mesh = plsc.VectorSubcoreMesh(core_axis_name="c", subcore_axis_name="s")
    return pl.kernel(
        _sc_chamfer,
        out_type=jax.ShapeDtypeStruct((_NW, 16), jnp.float32),
        mesh=mesh,
        scratch_types=[
            pltpu.VMEM((_N * 3,), jnp.float32),
            pltpu.VMEM((_N * 3,), jnp.float32),
            pltpu.VMEM((4, _N), jnp.float32),
            pltpu.VMEM((4, _N), jnp.float32),
            pltpu.VMEM((16,), jnp.float32),
        ],
    )(ab)


def _tc_body(ab_ref, out_ref):
    a = ab_ref[0, 0]
    b = ab_ref[1, 0]
    ab2 = lax.dot_general(a * -2.0, b, (((1,), (1,)), ((), ())),
                          preferred_element_type=jnp.float32)
    na = jnp.sum(a * a, axis=1)
    nb = jnp.sum(b * b, axis=1)
    d2 = (na[:, None] + ab2) + nb[None, :]
    m_b = jnp.min(d2, axis=0)
    m_a = jnp.min(d2, axis=1)
    loss = (jnp.mean(jnp.sqrt(jnp.maximum(m_b, 1e-12)))
            + jnp.mean(jnp.sqrt(jnp.maximum(m_a, 1e-12))))
    blk = jnp.full((1, 128), loss, jnp.float32)

    @pl.when(pl.program_id(0) == 0)
    def _():
        out_ref[...] = blk

    @pl.when(pl.program_id(0) != 0)
    def _():
        out_ref[...] = out_ref[...] + blk


def _tc_part(stacked):
    losses = pl.pallas_call(
        _tc_body,
        grid=(_B - _BSC,),
        in_specs=[
            pl.BlockSpec((2, 1, _N, 3), lambda i: (0, i + _BSC, 0, 0)),
        ],
        out_specs=pl.BlockSpec((1, 128), lambda i: (0, 0)),
        out_shape=jax.ShapeDtypeStruct((1, 128), jnp.float32),
    )(stacked)
    return losses[0, 0]


@jax.jit
def kernel(input, target):
    stacked = jnp.stack([input, target])
    sc_in = jnp.reshape(stacked[:, :_BSC], (2 * _BSC, _N * 3))
    sc_out = _sc_part(sc_in)
    tc_loss = _tc_part(stacked)
    return jnp.reshape(jnp.sum(sc_out) + tc_loss, (1,))
